# baseline (speedup 1.0000x reference)
import functools

import jax
import jax.numpy as jnp
from jax import lax
from jax.experimental import pallas as pl
from jax.experimental.pallas import tpu as pltpu
from jax.experimental.pallas import tpu_sc as plsc

_MAX_REL = 128
_L = 16
_NC = 2
_NS = 16
_NW = _NC * _NS
_Q = 4


def _body(x_hbm, tab_hbm, out_hbm, tabloc, x_v, ob, stg,
          sem_x, sem_s0, sem_s1, sem_d0, sem_d1, *, S, D, V, RG, JC):
    cid = lax.axis_index("c")
    sid = lax.axis_index("s")
    wid = cid * _NS + sid
    n_chunks = S // JC
    gpc = RG // _Q
    sem_s = (sem_s0, sem_s1)
    sem_d = (sem_d0, sem_d1)

    pltpu.sync_copy(tab_hbm, tabloc)

    def start_x(xp, jc):
        pltpu.make_async_copy(x_hbm.at[0, pl.ds(jc, JC)], x_v.at[xp],
                              sem_x).start()

    def wait_x(xp):
        pltpu.make_async_copy(x_hbm.at[0, pl.ds(0, JC)], x_v.at[xp],
                              sem_x).wait()

    def stream_q(p, q):
        return pltpu.make_async_copy(ob.at[p, q], stg.at[p, q, sid], sem_s[p])

    def dma_q(p, q, rblk, jc):
        return pltpu.make_async_copy(
            stg.at[p, q],
            out_hbm.at[0, pl.ds(32 * rblk + _NS * cid, _NS), pl.ds(jc, JC)],
            sem_d[p])

    def compute(p, xp, g_local, jc):
        rows = [32 * (g_local * _Q + q) + wid for q in range(_Q)]

        @plsc.parallel_loop(0, JC, unroll=4)
        def _t_body(t):
            idx = [jnp.clip(rows[q] - (jc + t), -_MAX_REL, _MAX_REL)
                   + _MAX_REL for q in range(_Q)]
            for kk in range(D // _L):
                sl = pl.ds(kk * _L, _L)
                xv = x_v[xp, t, sl]
                for q in range(_Q):
                    ob[p, q, t, sl] = tabloc[idx[q], sl] + xv

    def sync_and_fire(p, not_first, have_gm2, g_global, jc_cur):
        @pl.when(not_first)
        def _ws():
            for q in range(_Q):
                stream_q(1 - p, q).wait()

        @pl.when(jnp.logical_and(have_gm2, sid == 0))
        def _wd():
            for q in range(_Q):
                dma_q(p, q, 0, 0).wait()

        plsc.subcore_barrier()

        g_prev = g_global - 1
        c_prev = g_prev // gpc
        gl_prev = g_prev - c_prev * gpc
        jc_prev = c_prev * JC

        @pl.when(jnp.logical_and(not_first, sid == 0))
        def _fd():
            for q in range(_Q):
                dma_q(1 - p, q, gl_prev * _Q + q, jc_prev).start()

        for q in range(_Q):
            stream_q(p, q).start()

    start_x(0, 0)

    def chunk_pair(cc, carry):
        for cp in range(2):
            chunk = cc * 2 + cp
            jc = chunk * JC
            wait_x(cp)
            jc_next = jnp.minimum(jc + JC, S - JC)
            start_x(1 - cp, jc_next)

            def glp_body(glp, carry2):
                for p in range(2):
                    g_local = glp * 2 + p
                    g_global = chunk * gpc + g_local
                    first = jnp.logical_and(jnp.logical_and(cc == 0, glp == 0),
                                            (cp == 0) and (p == 0))
                    not_first = jnp.logical_not(first)
                    second = jnp.logical_and(
                        jnp.logical_and(cc == 0, glp == 0),
                        (cp == 0) and (p == 1))
                    have_gm2 = jnp.logical_not(jnp.logical_or(first, second))

                    compute(p, cp, g_local, jc)
                    sync_and_fire(p, not_first, have_gm2, g_global, jc)
                return carry2
            lax.fori_loop(0, gpc // 2, glp_body, 0)
        return carry
    lax.fori_loop(0, n_chunks // 2, chunk_pair, 0)

    for q in range(_Q):
        stream_q(1, q).wait()

    @pl.when(sid == 0)
    def _wd_last():
        for q in range(_Q):
            dma_q(0, q, 0, 0).wait()

    plsc.subcore_barrier()

    @pl.when(sid == 0)
    def _fd_last():
        for q in range(_Q):
            dma_q(1, q, (gpc - 1) * _Q + q, S - JC).start()
        for q in range(_Q):
            dma_q(1, q, 0, 0).wait()

    plsc.subcore_barrier()
    wait_x(0)


def kernel(x, rel_table):
    B, S, D = x.shape
    V = rel_table.shape[0]
    assert B == 1 and S % _NW == 0 and D % _L == 0
    RG = S // _NW
    JC = 32
    assert RG % (2 * _Q) == 0 and (S // JC) % 2 == 0

    mesh = plsc.VectorSubcoreMesh(core_axis_name="c", subcore_axis_name="s")
    body = functools.partial(_body, S=S, D=D, V=V, RG=RG, JC=JC)
    f = pl.kernel(
        body,
        out_type=jax.ShapeDtypeStruct((B, S, S, D), jnp.float32),
        scratch_types=[
            pltpu.VMEM((V, D), jnp.float32),
            pltpu.VMEM((2, JC, D), jnp.float32),
            pltpu.VMEM((2, _Q, JC, D), jnp.float32),
            pltpu.VMEM_SHARED((2, _Q, _NS, JC, D), jnp.float32),
            pltpu.SemaphoreType.DMA,
            pltpu.SemaphoreType.DMA,
            pltpu.SemaphoreType.DMA,
            pltpu.SemaphoreType.DMA,
            pltpu.SemaphoreType.DMA,
        ],
        mesh=mesh,
    )
    return f(x, rel_table)


# speedup vs baseline: 1.0031x; 1.0031x over previous
import functools

import jax
import jax.numpy as jnp
from jax import lax
from jax.experimental import pallas as pl
from jax.experimental.pallas import tpu as pltpu
from jax.experimental.pallas import tpu_sc as plsc

_MAX_REL = 128
_L = 16
_NC = 2
_NS = 16
_NW = _NC * _NS
_Q = 2


def _body(x_hbm, tab_hbm, out_hbm, tabloc, x_v, ob, stg,
          sem_x, sem_s0, sem_s1, sem_d0, sem_d1, *, S, D, V, RG, JC):
    cid = lax.axis_index("c")
    sid = lax.axis_index("s")
    wid = cid * _NS + sid
    n_chunks = S // JC
    gpc = RG // _Q
    sem_s = (sem_s0, sem_s1)
    sem_d = (sem_d0, sem_d1)

    pltpu.sync_copy(tab_hbm, tabloc)

    def start_x(xp, jc):
        pltpu.make_async_copy(x_hbm.at[0, pl.ds(jc, JC)], x_v.at[xp],
                              sem_x).start()

    def wait_x(xp):
        pltpu.make_async_copy(x_hbm.at[0, pl.ds(0, JC)], x_v.at[xp],
                              sem_x).wait()

    def stream_q(p, q):
        return pltpu.make_async_copy(ob.at[p, q], stg.at[p, q, sid], sem_s[p])

    def dma_q(p, q, rblk, jc):
        return pltpu.make_async_copy(
            stg.at[p, q],
            out_hbm.at[0, pl.ds(32 * rblk + _NS * cid, _NS), pl.ds(jc, JC)],
            sem_d[p])

    def compute(p, xp, g_local, jc):
        rows = [32 * (g_local * _Q + q) + wid for q in range(_Q)]

        @plsc.parallel_loop(0, JC, unroll=2)
        def _t_body(t):
            idx = [jnp.clip(rows[q] - (jc + t), -_MAX_REL, _MAX_REL)
                   + _MAX_REL for q in range(_Q)]
            for kk in range(D // _L):
                sl = pl.ds(kk * _L, _L)
                xv = x_v[xp, t, sl]
                for q in range(_Q):
                    ob[p, q, t, sl] = tabloc[idx[q], sl] + xv

    def sync_and_fire(p, not_first, have_gm2, g_global, jc_cur):
        @pl.when(not_first)
        def _ws():
            for q in range(_Q):
                stream_q(1 - p, q).wait()

        @pl.when(jnp.logical_and(have_gm2, sid == 0))
        def _wd():
            for q in range(_Q):
                dma_q(p, q, 0, 0).wait()

        plsc.subcore_barrier()

        g_prev = g_global - 1
        c_prev = g_prev // gpc
        gl_prev = g_prev - c_prev * gpc
        jc_prev = c_prev * JC

        @pl.when(jnp.logical_and(not_first, sid == 0))
        def _fd():
            for q in range(_Q):
                dma_q(1 - p, q, gl_prev * _Q + q, jc_prev).start()

        for q in range(_Q):
            stream_q(p, q).start()

    start_x(0, 0)

    def chunk_pair(cc, carry):
        for cp in range(2):
            chunk = cc * 2 + cp
            jc = chunk * JC
            wait_x(cp)
            jc_next = jnp.minimum(jc + JC, S - JC)
            start_x(1 - cp, jc_next)

            def glp_body(glp, carry2):
                for p in range(2):
                    g_local = glp * 2 + p
                    g_global = chunk * gpc + g_local
                    first = jnp.logical_and(jnp.logical_and(cc == 0, glp == 0),
                                            (cp == 0) and (p == 0))
                    not_first = jnp.logical_not(first)
                    second = jnp.logical_and(
                        jnp.logical_and(cc == 0, glp == 0),
                        (cp == 0) and (p == 1))
                    have_gm2 = jnp.logical_not(jnp.logical_or(first, second))

                    compute(p, cp, g_local, jc)
                    sync_and_fire(p, not_first, have_gm2, g_global, jc)
                return carry2
            lax.fori_loop(0, gpc // 2, glp_body, 0)
        return carry
    lax.fori_loop(0, n_chunks // 2, chunk_pair, 0)

    for q in range(_Q):
        stream_q(1, q).wait()

    @pl.when(sid == 0)
    def _wd_last():
        for q in range(_Q):
            dma_q(0, q, 0, 0).wait()

    plsc.subcore_barrier()

    @pl.when(sid == 0)
    def _fd_last():
        for q in range(_Q):
            dma_q(1, q, (gpc - 1) * _Q + q, S - JC).start()
        for q in range(_Q):
            dma_q(1, q, 0, 0).wait()

    plsc.subcore_barrier()
    wait_x(0)


def kernel(x, rel_table):
    B, S, D = x.shape
    V = rel_table.shape[0]
    assert B == 1 and S % _NW == 0 and D % _L == 0
    RG = S // _NW
    JC = 64
    assert RG % (2 * _Q) == 0 and (S // JC) % 2 == 0

    mesh = plsc.VectorSubcoreMesh(core_axis_name="c", subcore_axis_name="s")
    body = functools.partial(_body, S=S, D=D, V=V, RG=RG, JC=JC)
    f = pl.kernel(
        body,
        out_type=jax.ShapeDtypeStruct((B, S, S, D), jnp.float32),
        scratch_types=[
            pltpu.VMEM((V, D), jnp.float32),
            pltpu.VMEM((2, JC, D), jnp.float32),
            pltpu.VMEM((2, _Q, JC, D), jnp.float32),
            pltpu.VMEM_SHARED((2, _Q, _NS, JC, D), jnp.float32),
            pltpu.SemaphoreType.DMA,
            pltpu.SemaphoreType.DMA,
            pltpu.SemaphoreType.DMA,
            pltpu.SemaphoreType.DMA,
            pltpu.SemaphoreType.DMA,
        ],
        mesh=mesh,
    )
    return f(x, rel_table)


# speedup vs baseline: 1.0148x; 1.0117x over previous
import functools

import jax
import jax.numpy as jnp
from jax import lax
from jax.experimental import pallas as pl
from jax.experimental.pallas import tpu as pltpu
from jax.experimental.pallas import tpu_sc as plsc

_MAX_REL = 128
_L = 16
_NC = 2
_NS = 16
_NW = _NC * _NS
_Q = 4


def _body(x_hbm, tab_hbm, out_hbm, tabloc, x_v, ob, stg,
          sem_x, sem_s0, sem_s1, sem_d0, sem_d1, *, S, D, V, RG, JC):
    cid = lax.axis_index("c")
    sid = lax.axis_index("s")
    wid = cid * _NS + sid
    n_chunks = S // JC
    gpc = RG // _Q
    sem_s = (sem_s0, sem_s1)
    sem_d = (sem_d0, sem_d1)

    pltpu.sync_copy(tab_hbm, tabloc)

    def start_x(xp, jc):
        pltpu.make_async_copy(x_hbm.at[0, pl.ds(jc, JC)], x_v.at[xp],
                              sem_x).start()

    def wait_x(xp):
        pltpu.make_async_copy(x_hbm.at[0, pl.ds(0, JC)], x_v.at[xp],
                              sem_x).wait()

    def stream_q(p, q):
        return pltpu.make_async_copy(ob.at[p, q], stg.at[p, q, sid], sem_s[p])

    def dma_q(p, q, rblk, jc):
        return pltpu.make_async_copy(
            stg.at[p, q],
            out_hbm.at[0, pl.ds(32 * rblk + _NS * cid, _NS), pl.ds(jc, JC)],
            sem_d[p])

    def compute(p, xp, g_local, jc):
        rows = [32 * (g_local * _Q + q) + wid for q in range(_Q)]

        @plsc.parallel_loop(0, JC, unroll=2)
        def _t_body(t):
            idx = [jnp.clip(rows[q] - (jc + t), -_MAX_REL, _MAX_REL)
                   + _MAX_REL for q in range(_Q)]
            for kk in range(D // _L):
                sl = pl.ds(kk * _L, _L)
                xv = x_v[xp, t, sl]
                for q in range(_Q):
                    ob[p, q, t, sl] = tabloc[idx[q], sl] + xv

    def sync_and_fire(p, not_first, have_gm2, g_global, jc_cur):
        @pl.when(not_first)
        def _ws():
            for q in range(_Q):
                stream_q(1 - p, q).wait()

        @pl.when(jnp.logical_and(have_gm2, sid == 0))
        def _wd():
            for q in range(_Q):
                dma_q(p, q, 0, 0).wait()

        plsc.subcore_barrier()

        g_prev = g_global - 1
        c_prev = g_prev // gpc
        gl_prev = g_prev - c_prev * gpc
        jc_prev = c_prev * JC

        @pl.when(jnp.logical_and(not_first, sid == 0))
        def _fd():
            for q in range(_Q):
                dma_q(1 - p, q, gl_prev * _Q + q, jc_prev).start()

        for q in range(_Q):
            stream_q(p, q).start()

    start_x(0, 0)

    def chunk_pair(cc, carry):
        for cp in range(2):
            chunk = cc * 2 + cp
            jc = chunk * JC
            wait_x(cp)
            jc_next = jnp.minimum(jc + JC, S - JC)
            start_x(1 - cp, jc_next)

            def glp_body(glp, carry2):
                for p in range(2):
                    g_local = glp * 2 + p
                    g_global = chunk * gpc + g_local
                    first = jnp.logical_and(jnp.logical_and(cc == 0, glp == 0),
                                            (cp == 0) and (p == 0))
                    not_first = jnp.logical_not(first)
                    second = jnp.logical_and(
                        jnp.logical_and(cc == 0, glp == 0),
                        (cp == 0) and (p == 1))
                    have_gm2 = jnp.logical_not(jnp.logical_or(first, second))

                    compute(p, cp, g_local, jc)
                    sync_and_fire(p, not_first, have_gm2, g_global, jc)
                return carry2
            lax.fori_loop(0, gpc // 2, glp_body, 0)
        return carry
    lax.fori_loop(0, n_chunks // 2, chunk_pair, 0)

    for q in range(_Q):
        stream_q(1, q).wait()

    @pl.when(sid == 0)
    def _wd_last():
        for q in range(_Q):
            dma_q(0, q, 0, 0).wait()

    plsc.subcore_barrier()

    @pl.when(sid == 0)
    def _fd_last():
        for q in range(_Q):
            dma_q(1, q, (gpc - 1) * _Q + q, S - JC).start()
        for q in range(_Q):
            dma_q(1, q, 0, 0).wait()

    plsc.subcore_barrier()
    wait_x(0)


def kernel(x, rel_table):
    B, S, D = x.shape
    V = rel_table.shape[0]
    assert B == 1 and S % _NW == 0 and D % _L == 0
    RG = S // _NW
    JC = 32
    assert RG % (2 * _Q) == 0 and (S // JC) % 2 == 0

    mesh = plsc.VectorSubcoreMesh(core_axis_name="c", subcore_axis_name="s")
    body = functools.partial(_body, S=S, D=D, V=V, RG=RG, JC=JC)
    f = pl.kernel(
        body,
        out_type=jax.ShapeDtypeStruct((B, S, S, D), jnp.float32),
        scratch_types=[
            pltpu.VMEM((V, D), jnp.float32),
            pltpu.VMEM((2, JC, D), jnp.float32),
            pltpu.VMEM((2, _Q, JC, D), jnp.float32),
            pltpu.VMEM_SHARED((2, _Q, _NS, JC, D), jnp.float32),
            pltpu.SemaphoreType.DMA,
            pltpu.SemaphoreType.DMA,
            pltpu.SemaphoreType.DMA,
            pltpu.SemaphoreType.DMA,
            pltpu.SemaphoreType.DMA,
        ],
        mesh=mesh,
    )
    return f(x, rel_table)
